# baseline (speedup 1.0000x reference)
import functools

import jax
import jax.numpy as jnp
from jax import lax
from jax.experimental import pallas as pl
from jax.experimental.pallas import tpu as pltpu
from jax.experimental.pallas import tpu_sc as plsc

D_MODEL = 1024
N_EXP = 8
TOPK = 2
GATE_BLOCK = 512
ROW_BLOCK = 256
NC, NS = 2, 16
NW = NC * NS
GCH = 16
CCH = 16


def _gating_body(x_ref, gw_ref, gb_ref, idx_ref, w_ref):
    x = x_ref[...]
    logits = jax.lax.dot_general(
        x, gw_ref[...], (((1,), (1,)), ((), ())),
        preferred_element_type=jnp.float32) + gb_ref[...]
    iota = jax.lax.broadcasted_iota(jnp.int32, logits.shape, 1)
    m1 = jnp.max(logits, axis=1, keepdims=True)
    i1 = jnp.min(jnp.where(logits == m1, iota, N_EXP), axis=1, keepdims=True)
    l2 = jnp.where(iota == i1, -1e30, logits)
    m2 = jnp.max(l2, axis=1, keepdims=True)
    i2 = jnp.min(jnp.where(l2 == m2, iota, N_EXP), axis=1, keepdims=True)
    e2 = jnp.exp(m2 - m1)
    w1 = 1.0 / (1.0 + e2)
    w2 = e2 / (1.0 + e2)
    idx_ref[...] = jnp.concatenate([i1, i2], axis=1)
    w_ref[...] = jnp.concatenate([w1, w2], axis=1)


def _gating(xf, gate_w, gate_b):
    n = xf.shape[0]
    return pl.pallas_call(
        _gating_body,
        grid=(n // GATE_BLOCK,),
        in_specs=[
            pl.BlockSpec((GATE_BLOCK, D_MODEL), lambda i: (i, 0)),
            pl.BlockSpec((N_EXP, D_MODEL), lambda i: (0, 0)),
            pl.BlockSpec((1, N_EXP), lambda i: (0, 0)),
        ],
        out_specs=[
            pl.BlockSpec((GATE_BLOCK, TOPK), lambda i: (i, 0)),
            pl.BlockSpec((GATE_BLOCK, TOPK), lambda i: (i, 0)),
        ],
        out_shape=[
            jax.ShapeDtypeStruct((n, TOPK), jnp.int32),
            jax.ShapeDtypeStruct((n, TOPK), jnp.float32),
        ],
    )(xf, gate_w, gate_b.reshape(1, N_EXP))


def _route(idx, n_blocks):
    e_flat = idx.reshape(-1)
    p = e_flat.shape[0]
    sub = 128
    oh = (e_flat.reshape(p // sub, sub, 1)
          == jnp.arange(N_EXP)[None, None, :]).astype(jnp.int32)
    within = jnp.cumsum(oh, axis=1)
    sub_tot = within[:, -1, :]
    sub_pre = jnp.cumsum(sub_tot, axis=0) - sub_tot
    ranks_all = within - oh + sub_pre[:, None, :]
    rank = jnp.sum(ranks_all * oh, axis=2).reshape(p)
    counts = sub_tot.sum(axis=0)
    blocks_per_e = (counts + ROW_BLOCK - 1) // ROW_BLOCK
    block_end = jnp.cumsum(blocks_per_e)
    pad_off = (block_end - blocks_per_e) * ROW_BLOCK
    pos = pad_off[e_flat] + rank
    block_expert = jnp.minimum(
        jnp.searchsorted(block_end, jnp.arange(n_blocks), side="right"),
        N_EXP - 1).astype(jnp.int32)
    return pos, block_expert


def _sc_gather(xf, pos, bp):
    mesh = plsc.VectorSubcoreMesh(core_axis_name="c", subcore_axis_name="s")
    p_tot = pos.shape[0]
    per_w = p_tot // NW
    n_ch = per_w // GCH

    @functools.partial(
        pl.kernel, mesh=mesh,
        out_type=jax.ShapeDtypeStruct((bp, D_MODEL), jnp.float32),
        scratch_types=[
            pltpu.VMEM((per_w,), jnp.int32),
            pltpu.VMEM((GCH, D_MODEL), jnp.float32),
            pltpu.VMEM((GCH, D_MODEL), jnp.float32),
            pltpu.SemaphoreType.DMA,
            pltpu.SemaphoreType.DMA,
            pltpu.SemaphoreType.DMA,
            pltpu.SemaphoreType.DMA,
        ],
    )
    def gather_k(x_hbm, pos_hbm, out_hbm, pos_v, buf0, buf1,
                 gs0, gs1, ws0, ws1):
        wid = lax.axis_index("s") * NC + lax.axis_index("c")
        base = wid * per_w
        pltpu.sync_copy(pos_hbm.at[pl.ds(base, per_w)], pos_v)
        bufs, gsems, wsems = (buf0, buf1), (gs0, gs1), (ws0, ws1)
        lane = lax.iota(jnp.int32, 16)

        def start_gather(c):
            s = c % 2
            tok = lax.shift_right_logical(lane + (base + c * GCH), 1)
            return pltpu.async_copy(x_hbm.at[tok], bufs[s], gsems[s])

        wpend = [None, None]
        gpend = [start_gather(0), None]
        for c in range(n_ch):
            s = c % 2
            if c + 1 < n_ch:
                if wpend[1 - s] is not None:
                    wpend[1 - s].wait()
                    wpend[1 - s] = None
                gpend[1 - s] = start_gather(c + 1)
            gpend[s].wait()
            slot = pos_v[pl.ds(c * GCH, GCH)]
            wpend[s] = pltpu.async_copy(bufs[s], out_hbm.at[slot], wsems[s])
        for s in range(2):
            if wpend[s] is not None:
                wpend[s].wait()

    return gather_k(xf, pos)


def _gmm_body(be_ref, xg_ref, w_ref, b_ref, o_ref):
    o_ref[...] = jax.lax.dot_general(
        xg_ref[...], w_ref[0], (((1,), (1,)), ((), ())),
        preferred_element_type=jnp.float32) + b_ref[0]


def _grouped_matmul(xg, expert_w, expert_b, block_expert, n_blocks):
    bp = xg.shape[0]
    grid_spec = pltpu.PrefetchScalarGridSpec(
        num_scalar_prefetch=1,
        grid=(n_blocks,),
        in_specs=[
            pl.BlockSpec((ROW_BLOCK, D_MODEL), lambda b, be: (b, 0)),
            pl.BlockSpec((1, D_MODEL, D_MODEL), lambda b, be: (be[b], 0, 0)),
            pl.BlockSpec((1, 1, D_MODEL), lambda b, be: (be[b], 0, 0)),
        ],
        out_specs=pl.BlockSpec((ROW_BLOCK, D_MODEL), lambda b, be: (b, 0)),
    )
    return pl.pallas_call(
        _gmm_body,
        grid_spec=grid_spec,
        out_shape=jax.ShapeDtypeStruct((bp, D_MODEL), jnp.float32),
    )(block_expert, xg, expert_w, expert_b.reshape(N_EXP, 1, D_MODEL))


def _sc_combine(y, inv0, inv1, wa, wb, n):
    mesh = plsc.VectorSubcoreMesh(core_axis_name="c", subcore_axis_name="s")
    per_w = n // NW
    n_ch = per_w // CCH
    unroll = 4
    n_sl = D_MODEL // 16

    @functools.partial(
        pl.kernel, mesh=mesh,
        out_type=jax.ShapeDtypeStruct((n, D_MODEL), jnp.float32),
        scratch_types=[
            pltpu.VMEM((per_w,), jnp.int32),
            pltpu.VMEM((per_w,), jnp.int32),
            pltpu.VMEM((per_w,), jnp.float32),
            pltpu.VMEM((per_w,), jnp.float32),
            pltpu.VMEM((CCH, D_MODEL), jnp.float32),
            pltpu.VMEM((CCH, D_MODEL), jnp.float32),
            pltpu.VMEM((CCH, D_MODEL), jnp.float32),
            pltpu.VMEM((CCH, D_MODEL), jnp.float32),
            pltpu.SemaphoreType.DMA,
            pltpu.SemaphoreType.DMA,
        ],
    )
    def combine_k(y_hbm, i0_hbm, i1_hbm, wa_hbm, wb_hbm, out_hbm,
                  ia_v, ib_v, wa_v, wb_v, ra0, rb0, ra1, rb1, sem0, sem1):
        wid = lax.axis_index("s") * NC + lax.axis_index("c")
        base = wid * per_w
        pltpu.sync_copy(i0_hbm.at[pl.ds(base, per_w)], ia_v)
        pltpu.sync_copy(i1_hbm.at[pl.ds(base, per_w)], ib_v)
        pltpu.sync_copy(wa_hbm.at[pl.ds(base, per_w)], wa_v)
        pltpu.sync_copy(wb_hbm.at[pl.ds(base, per_w)], wb_v)
        ras, rbs, sems = (ra0, ra1), (rb0, rb1), (sem0, sem1)

        def start(c):
            s = c % 2
            da = pltpu.async_copy(
                y_hbm.at[ia_v.at[pl.ds(c * CCH, CCH)]], ras[s], sems[s])
            db = pltpu.async_copy(
                y_hbm.at[ib_v.at[pl.ds(c * CCH, CCH)]], rbs[s], sems[s])
            return da, db

        pend = [start(0), None]
        for c in range(n_ch):
            if c + 1 < n_ch:
                pend[(c + 1) % 2] = start(c + 1)
            da, db = pend[c % 2]
            da.wait()
            db.wait()
            s = c % 2
            ra, rb = ras[s], rbs[s]
            wav = wa_v[pl.ds(c * CCH, CCH)]
            wbv = wb_v[pl.ds(c * CCH, CCH)]
            for i in range(CCH):
                w0 = jnp.full((16,), wav[i], jnp.float32)
                w1 = jnp.full((16,), wbv[i], jnp.float32)

                def row_body(t, carry, ra=ra, rb=rb, i=i, w0=w0, w1=w1):
                    for u in range(unroll):
                        j = t * unroll + u
                        sl = pl.ds(j * 16, 16)
                        ra[i, sl] = ra[i, sl] * w0 + rb[i, sl] * w1
                    return carry

                lax.fori_loop(0, n_sl // unroll, row_body, 0)
            pltpu.sync_copy(ra, out_hbm.at[pl.ds(base + c * CCH, CCH)])

    return combine_k(y, inv0, inv1, wa, wb)


def kernel(x, gate_w, gate_b, expert_w, expert_b):
    batch, seq, d = x.shape
    xf = x.reshape(-1, d)
    n = xf.shape[0]
    n_blocks = (n * TOPK) // ROW_BLOCK + N_EXP
    bp = n_blocks * ROW_BLOCK

    idx, w = _gating(xf, gate_w, gate_b)
    pos, block_expert = _route(idx, n_blocks)
    xg = _sc_gather(xf, pos, bp)
    y = _grouped_matmul(xg, expert_w, expert_b, block_expert, n_blocks)
    out = _sc_combine(y, pos[0::TOPK], pos[1::TOPK], w[:, 0], w[:, 1], n)
    return out.reshape(batch, seq, d)
